# Initial kernel scaffold; baseline (speedup 1.0000x reference)
import jax
import jax.numpy as jnp
from jax.experimental import pallas as pl


def kernel(xyr01, msg_edge_index, cand_pairs_uv, node_in_w1, node_in_b1, node_in_w2, node_in_b2, phi_m_w1, phi_m_b1, phi_m_w2, phi_m_b2, phi_h_w1, phi_h_b1, phi_h_w2, phi_h_b2, ln_g, ln_b, eh_w1, eh_b1, eh_w2, eh_b2):
    raise NotImplementedError("write your pallas kernel here")


# speedup vs baseline: 14.1742x; 14.1742x over previous
import functools

import jax
import jax.numpy as jnp
from jax import lax
from jax.experimental import pallas as pl
from jax.experimental.pallas import tpu as pltpu
from jax.experimental.pallas import tpu_sc as plsc

N = 10000
E = 320000
P = 320000
DH = 128
NRBF = 16
RMAX = 1.4142135623730951
INV2S2 = 56.25

NPAD = 10240
TL = 256
NW = 32
CH = 128
NCH = 80
CHG = 64
NCHG = 160
EPW = CH * NCH
EPAD = NW * EPW
NACC = 10112
BN = 1024
BE = 2048


@functools.cache
def _get_sc_gather2():
    mesh = plsc.VectorSubcoreMesh(core_axis_name="c", subcore_axis_name="s")

    @functools.partial(
        pl.kernel,
        mesh=mesh,
        out_type=(jax.ShapeDtypeStruct((EPAD, TL), jnp.float32),
                  jax.ShapeDtypeStruct((EPAD, TL), jnp.float32)),
        scratch_types=[
            pltpu.VMEM((NCHG, CHG), jnp.int32),
            pltpu.VMEM((NCHG, CHG), jnp.int32),
            pltpu.VMEM((CHG, TL), jnp.float32),
            pltpu.VMEM((CHG, TL), jnp.float32),
            pltpu.VMEM((CHG, TL), jnp.float32),
            pltpu.VMEM((CHG, TL), jnp.float32),
            pltpu.SemaphoreType.DMA,
            pltpu.SemaphoreType.DMA,
            pltpu.SemaphoreType.DMA,
            pltpu.SemaphoreType.DMA,
        ],
    )
    def sc_gather2(tab_a, tab_b, idx_a, idx_b, out_a, out_b,
                   ia_v, ib_v, buf_a0, buf_a1, buf_b0, buf_b1,
                   s_a0, s_a1, s_b0, s_b1):
        wid = lax.axis_index("s") * 2 + lax.axis_index("c")
        pltpu.sync_copy(idx_a.at[wid], ia_v)
        pltpu.sync_copy(idx_b.at[wid], ib_v)
        buf_a = (buf_a0, buf_a1)
        buf_b = (buf_b0, buf_b1)
        sem_a = (s_a0, s_a1)
        sem_b = (s_b0, s_b1)
        base0 = wid * EPW

        def start(c, b):
            pltpu.async_copy(tab_a.at[ia_v.at[c]], buf_a[b], sem_a[b])
            pltpu.async_copy(tab_b.at[ib_v.at[c]], buf_b[b], sem_b[b])

        def finish(c, b):
            pltpu.make_async_copy(tab_a.at[ia_v.at[c]], buf_a[b], sem_a[b]).wait()
            pltpu.make_async_copy(tab_b.at[ib_v.at[c]], buf_b[b], sem_b[b]).wait()
            pltpu.sync_copy(buf_a[b], out_a.at[pl.ds(base0 + c * CHG, CHG)])
            pltpu.sync_copy(buf_b[b], out_b.at[pl.ds(base0 + c * CHG, CHG)])

        start(0, 0)

        @pl.loop(0, NCHG, step=2)
        def _(c):
            start(c + 1, 1)
            finish(c, 0)

            @pl.when(c + 2 < NCHG)
            def _():
                start(c + 2, 0)

            finish(c + 1, 1)

    return sc_gather2


def _sc_gather2(tab_a, tab_b, idx_a, idx_b):
    return _get_sc_gather2()(tab_a, tab_b, idx_a, idx_b)


@functools.cache
def _get_sc_scatter():
    mesh = plsc.VectorSubcoreMesh(core_axis_name="c", subcore_axis_name="s")

    @functools.partial(
        pl.kernel,
        mesh=mesh,
        out_type=jax.ShapeDtypeStruct((2, NACC, DH), jnp.float32),
        scratch_types=[
            pltpu.VMEM((NCH, CH), jnp.int32),
            pltpu.VMEM((CH, DH), jnp.float32),
            pltpu.VMEM((CH, DH), jnp.float32),
            pltpu.VMEM_SHARED((NACC, DH), jnp.float32),
            pltpu.SemaphoreType.DMA,
            pltpu.SemaphoreType.DMA,
        ],
    )
    def sc_scatter(e_hbm, idx_d, z128, agg_out,
                   id_v, eb0, eb1, acc, s0, s1):
        cid = lax.axis_index("c")
        sid = lax.axis_index("s")
        wid = sid * 2 + cid
        rows = NACC // 16
        pltpu.sync_copy(idx_d.at[wid], id_v)
        pltpu.sync_copy(z128, acc.at[pl.ds(sid * rows, rows)])
        plsc.subcore_barrier()

        base0 = wid * EPW
        ebuf = (eb0, eb1)
        sems = (s0, s1)

        def start(c, b):
            pltpu.async_copy(e_hbm.at[pl.ds(base0 + c * CH, CH)], ebuf[b],
                             sems[b])

        def finish(c, b):
            pltpu.make_async_copy(
                e_hbm.at[pl.ds(base0 + c * CH, CH)], ebuf[b], sems[b]).wait()
            pltpu.sync_copy(ebuf[b], acc.at[id_v.at[c]], add=True)

        start(0, 0)

        @pl.loop(0, NCH, step=2)
        def _(c):
            start(c + 1, 1)
            finish(c, 0)

            @pl.when(c + 2 < NCH)
            def _():
                start(c + 2, 0)

            finish(c + 1, 1)

        plsc.subcore_barrier()
        pltpu.sync_copy(acc.at[pl.ds(sid * rows, rows)],
                        agg_out.at[cid, pl.ds(sid * rows, rows)])

    return sc_scatter


def _sc_scatter(e_hbm, idx_d, z128):
    return _get_sc_scatter()(e_hbm, idx_d, z128)


@functools.cache
def _get_sc_degree():
    mesh = plsc.VectorSubcoreMesh(core_axis_name="c", subcore_axis_name="s")

    @functools.partial(
        pl.kernel,
        mesh=mesh,
        out_type=jax.ShapeDtypeStruct((2, NACC, 16), jnp.float32),
        scratch_types=[
            pltpu.VMEM((NCH, CH), jnp.int32),
            pltpu.VMEM((CH, 16), jnp.float32),
            pltpu.VMEM_SHARED((NACC, 16), jnp.float32),
        ],
    )
    def sc_degree(idx_d, z16, deg_out, id_v, ones_v, dacc):
        cid = lax.axis_index("c")
        sid = lax.axis_index("s")
        wid = sid * 2 + cid
        rows = NACC // 16
        pltpu.sync_copy(idx_d.at[wid], id_v)
        pltpu.sync_copy(z16, dacc.at[pl.ds(sid * rows, rows)])

        @pl.loop(0, CH)
        def _(i):
            ones_v[i, :] = jnp.ones((16,), jnp.float32)

        plsc.subcore_barrier()

        @pl.loop(0, NCH)
        def _(c):
            pltpu.sync_copy(ones_v, dacc.at[id_v.at[c]], add=True)

        plsc.subcore_barrier()
        pltpu.sync_copy(dacc.at[pl.ds(sid * rows, rows)],
                        deg_out.at[cid, pl.ds(sid * rows, rows)])

    return sc_degree


def _sc_degree(idx_d, z16):
    return _get_sc_degree()(idx_d, z16)


def _prologue_body(x_ref, w1_ref, b1_ref, w2_ref, b2_ref, wa_ref, wb_ref,
                   b1m_ref, h_ref, ts_ref, td_ref):
    x = x_ref[...]
    h1 = jnp.maximum(
        jnp.dot(x, w1_ref[...], preferred_element_type=jnp.float32)
        + b1_ref[...], 0.0)
    h = jnp.dot(h1, w2_ref[...], preferred_element_type=jnp.float32) + b2_ref[...]
    h_ref[...] = h
    zpad = jnp.zeros((x.shape[0], TL - DH - 3), jnp.float32)
    ts = jnp.dot(h, wa_ref[...], preferred_element_type=jnp.float32)
    td = (jnp.dot(h, wb_ref[...], preferred_element_type=jnp.float32)
          + b1m_ref[...])
    ts_ref[...] = jnp.concatenate([ts, x, zpad], axis=1)
    td_ref[...] = jnp.concatenate([td, x, zpad], axis=1)


def _edge_body(mu_ref, w1c_ref, hs_ref, hd_ref, e_ref):
    hs = hs_ref[...]
    hd = hd_ref[...]
    dx0 = hs[:, 128:129] - hd[:, 128:129]
    dx1 = hs[:, 129:130] - hd[:, 129:130]
    r = jnp.sqrt(dx0 * dx0 + dx1 * dx1 + 1e-8)
    t = r - mu_ref[...]
    rbf = jnp.exp(t * t * (-INV2S2))
    g = jnp.dot(rbf, w1c_ref[...], preferred_element_type=jnp.float32)
    e_ref[...] = jnp.maximum(hs[:, 0:DH] + hd[:, 0:DH] + g, 0.0)


def _node_body(x_ref, h_ref, a0_ref, a1_ref, d0_ref, d1_ref,
               pmw2_ref, pmb2_ref, pw1a_ref, pw1b_ref, pb1_ref,
               pw2_ref, pb2_ref, lng_ref, lnb_ref, ea_ref, eb_ref, eb1_ref,
               au_ref, av_ref):
    h = h_ref[...]
    agg = jnp.dot(a0_ref[...] + a1_ref[...], pmw2_ref[...],
                  preferred_element_type=jnp.float32)
    deg = d0_ref[:, 0:1] + d1_ref[:, 0:1]
    agg = agg + deg * pmb2_ref[...]
    t1 = jnp.maximum(
        jnp.dot(h, pw1a_ref[...], preferred_element_type=jnp.float32)
        + jnp.dot(agg, pw1b_ref[...], preferred_element_type=jnp.float32)
        + pb1_ref[...], 0.0)
    h_up = jnp.dot(t1, pw2_ref[...], preferred_element_type=jnp.float32) + pb2_ref[...]
    y = h + h_up
    mu = jnp.mean(y, axis=1, keepdims=True)
    var = jnp.mean((y - mu) ** 2, axis=1, keepdims=True)
    hn = (y - mu) * lax.rsqrt(var + 1e-5) * lng_ref[...] + lnb_ref[...]
    x = x_ref[...]
    zpad = jnp.zeros((x.shape[0], TL - DH - 3), jnp.float32)
    au = jnp.dot(hn, ea_ref[...], preferred_element_type=jnp.float32)
    av = jnp.dot(hn, eb_ref[...], preferred_element_type=jnp.float32) + eb1_ref[...]
    au_ref[...] = jnp.concatenate([au, x, zpad], axis=1)
    av_ref[...] = jnp.concatenate([av, x, zpad], axis=1)


def _pair_body(w4_ref, w2r_ref, b2_ref, pu_ref, pv_ref, o_ref):
    pu = pu_ref[...]
    pv = pv_ref[...]
    dx0 = pu[:, 128:129] - pv[:, 128:129]
    dx1 = pu[:, 129:130] - pv[:, 129:130]
    dist = jnp.sqrt(dx0 * dx0 + dx1 * dx1 + 1e-8)
    ru = pu[:, 130:131]
    rv = pv[:, 130:131]
    w4 = w4_ref[...]
    g = (ru * w4[0:1, :] + rv * w4[1:2, :] + dist * w4[2:3, :]
         + jnp.abs(ru - rv) * w4[3:4, :])
    z = jnp.maximum(pu[:, 0:DH] + pv[:, 0:DH] + g, 0.0)
    o_ref[...] = jnp.sum(z * w2r_ref[...], axis=1, keepdims=True) + b2_ref[...]


def _sds(shape):
    return jax.ShapeDtypeStruct(shape, jnp.float32)


_prologue = pl.pallas_call(
    _prologue_body,
    grid=(NPAD // BN,),
    in_specs=[
        pl.BlockSpec((BN, 3), lambda i: (i, 0)),
        pl.BlockSpec((3, DH), lambda i: (0, 0)),
        pl.BlockSpec((1, DH), lambda i: (0, 0)),
        pl.BlockSpec((DH, DH), lambda i: (0, 0)),
        pl.BlockSpec((1, DH), lambda i: (0, 0)),
        pl.BlockSpec((DH, DH), lambda i: (0, 0)),
        pl.BlockSpec((DH, DH), lambda i: (0, 0)),
        pl.BlockSpec((1, DH), lambda i: (0, 0)),
    ],
    out_specs=[
        pl.BlockSpec((BN, DH), lambda i: (i, 0)),
        pl.BlockSpec((BN, TL), lambda i: (i, 0)),
        pl.BlockSpec((BN, TL), lambda i: (i, 0)),
    ],
    out_shape=(_sds((NPAD, DH)), _sds((NPAD, TL)), _sds((NPAD, TL))),
)

_edge_mlp = pl.pallas_call(
    _edge_body,
    grid=(EPAD // BE,),
    in_specs=[
        pl.BlockSpec((1, NRBF), lambda i: (0, 0)),
        pl.BlockSpec((NRBF, DH), lambda i: (0, 0)),
        pl.BlockSpec((BE, TL), lambda i: (i, 0)),
        pl.BlockSpec((BE, TL), lambda i: (i, 0)),
    ],
    out_specs=pl.BlockSpec((BE, DH), lambda i: (i, 0)),
    out_shape=_sds((EPAD, DH)),
)

_node_update = pl.pallas_call(
    _node_body,
    grid=(NPAD // BN,),
    in_specs=[
        pl.BlockSpec((BN, 3), lambda i: (i, 0)),
        pl.BlockSpec((BN, DH), lambda i: (i, 0)),
        pl.BlockSpec((BN, DH), lambda i: (i, 0)),
        pl.BlockSpec((BN, DH), lambda i: (i, 0)),
        pl.BlockSpec((BN, 16), lambda i: (i, 0)),
        pl.BlockSpec((BN, 16), lambda i: (i, 0)),
        pl.BlockSpec((DH, DH), lambda i: (0, 0)),
        pl.BlockSpec((1, DH), lambda i: (0, 0)),
        pl.BlockSpec((DH, DH), lambda i: (0, 0)),
        pl.BlockSpec((DH, DH), lambda i: (0, 0)),
        pl.BlockSpec((1, DH), lambda i: (0, 0)),
        pl.BlockSpec((DH, DH), lambda i: (0, 0)),
        pl.BlockSpec((1, DH), lambda i: (0, 0)),
        pl.BlockSpec((1, DH), lambda i: (0, 0)),
        pl.BlockSpec((1, DH), lambda i: (0, 0)),
        pl.BlockSpec((DH, DH), lambda i: (0, 0)),
        pl.BlockSpec((DH, DH), lambda i: (0, 0)),
        pl.BlockSpec((1, DH), lambda i: (0, 0)),
    ],
    out_specs=[
        pl.BlockSpec((BN, TL), lambda i: (i, 0)),
        pl.BlockSpec((BN, TL), lambda i: (i, 0)),
    ],
    out_shape=(_sds((NPAD, TL)), _sds((NPAD, TL))),
)

_pair_head = pl.pallas_call(
    _pair_body,
    grid=(EPAD // BE,),
    in_specs=[
        pl.BlockSpec((4, DH), lambda i: (0, 0)),
        pl.BlockSpec((1, DH), lambda i: (0, 0)),
        pl.BlockSpec((1, 1), lambda i: (0, 0)),
        pl.BlockSpec((BE, TL), lambda i: (i, 0)),
        pl.BlockSpec((BE, TL), lambda i: (i, 0)),
    ],
    out_specs=pl.BlockSpec((BE, 1), lambda i: (i, 0)),
    out_shape=_sds((EPAD, 1)),
)


def kernel(xyr01, msg_edge_index, cand_pairs_uv,
           node_in_w1, node_in_b1, node_in_w2, node_in_b2,
           phi_m_w1, phi_m_b1, phi_m_w2, phi_m_b2,
           phi_h_w1, phi_h_b1, phi_h_w2, phi_h_b2,
           ln_g, ln_b,
           eh_w1, eh_b1, eh_w2, eh_b2):
    f32 = jnp.float32
    i32 = jnp.int32
    x = xyr01.astype(f32)
    xp = jnp.zeros((NPAD, 3), f32).at[:N].set(x)

    src = msg_edge_index[0].astype(i32)
    dst = msg_edge_index[1].astype(i32)
    u = cand_pairs_uv[:, 0].astype(i32)
    v = cand_pairs_uv[:, 1].astype(i32)
    pad = EPAD - E
    zpad_i = jnp.zeros((pad,), i32)
    srcp = jnp.concatenate([src, zpad_i]).reshape(NW, NCHG, CHG)
    dstp = jnp.concatenate([dst, jnp.full((pad,), N, i32)]).reshape(NW, NCHG, CHG)
    dstp_s = dstp.reshape(NW, NCH, CH)
    up = jnp.concatenate([u, zpad_i]).reshape(NW, NCHG, CHG)
    vp = jnp.concatenate([v, zpad_i]).reshape(NW, NCHG, CHG)

    wa = phi_m_w1[0:DH]
    wb = phi_m_w1[DH:2 * DH]
    w1c = phi_m_w1[2 * DH:]
    mu_row = jnp.linspace(0.0, RMAX, NRBF, dtype=f32).reshape(1, NRBF)

    h, ts_tab, td_tab = _prologue(
        xp, node_in_w1, node_in_b1.reshape(1, DH), node_in_w2,
        node_in_b2.reshape(1, DH), wa, wb, phi_m_b1.reshape(1, DH))

    hs, hd = _sc_gather2(ts_tab, td_tab, srcp, dstp)

    e = _edge_mlp(mu_row, w1c, hs, hd)

    z128 = jnp.zeros((NACC // 16, DH), f32)
    z16 = jnp.zeros((NACC // 16, 16), f32)
    agg2 = _sc_scatter(e, dstp_s, z128)
    deg2 = _sc_degree(dstp_s, z16)

    npad2 = ((0, 0), (0, NPAD - NACC), (0, 0))
    agg2 = jnp.pad(agg2, npad2)
    deg2 = jnp.pad(deg2, npad2)

    au_tab, av_tab = _node_update(
        xp, h, agg2[0], agg2[1], deg2[0], deg2[1],
        phi_m_w2, phi_m_b2.reshape(1, DH),
        phi_h_w1[0:DH], phi_h_w1[DH:2 * DH], phi_h_b1.reshape(1, DH),
        phi_h_w2, phi_h_b2.reshape(1, DH),
        ln_g.reshape(1, DH), ln_b.reshape(1, DH),
        eh_w1[0:DH], eh_w1[DH:2 * DH], eh_b1.reshape(1, DH))

    pu, pv = _sc_gather2(au_tab, av_tab, up, vp)

    out = _pair_head(eh_w1[2 * DH:], eh_w2.reshape(1, DH),
                     eh_b2.reshape(1, 1), pu, pv)
    return out[:P, 0]


# speedup vs baseline: 18.6509x; 1.3158x over previous
import dataclasses
import functools

import jax
import jax.numpy as jnp
from jax import lax
from jax.experimental import pallas as pl
from jax.experimental.pallas import tpu as pltpu
from jax.experimental.pallas import tpu_sc as plsc

N = 10000
E = 320000
P = 320000
DH = 128
NRBF = 16
RMAX = 1.4142135623730951
C2S2 = 2.0 * (RMAX / 15.0) * (RMAX / 15.0)
INV2S2 = 56.25

NPAD = 10240
NW = 32
CH = 128
NCH = 80
EPW = CH * NCH
EPAD = NW * EPW
NACC = 10112
BN = 1024
BE = 2048
NBE = EPAD // BE
GPB = BE // CH

_P_HI = None


def _sc_compiler_params():
    cp = pltpu.CompilerParams()
    if "needs_layout_passes" in pltpu.CompilerParams.__dataclass_fields__:
        cp = dataclasses.replace(cp, needs_layout_passes=False)
    return cp


@functools.cache
def _get_sc_gather2():
    mesh = plsc.VectorSubcoreMesh(core_axis_name="c", subcore_axis_name="s")

    @functools.partial(
        pl.kernel,
        mesh=mesh,
        compiler_params=_sc_compiler_params(),
        out_type=(jax.ShapeDtypeStruct((EPAD, DH), jnp.float32),
                  jax.ShapeDtypeStruct((EPAD, DH), jnp.float32),
                  jax.ShapeDtypeStruct((NBE * 8 * BE,), jnp.float32)),
        scratch_types=[
            pltpu.VMEM((NCH, CH), jnp.int32),
            pltpu.VMEM((NCH, CH), jnp.int32),
            pltpu.VMEM((CH, DH), jnp.float32),
            pltpu.VMEM((CH, DH), jnp.float32),
            pltpu.VMEM((CH, DH), jnp.float32),
            pltpu.VMEM((CH, DH), jnp.float32),
            pltpu.VMEM((NPAD,), jnp.float32),
            pltpu.VMEM((NPAD,), jnp.float32),
            pltpu.VMEM((NPAD,), jnp.float32),
            pltpu.VMEM((CH,), jnp.float32),
            pltpu.VMEM((CH,), jnp.float32),
            pltpu.VMEM((CH,), jnp.float32),
            pltpu.SemaphoreType.DMA,
            pltpu.SemaphoreType.DMA,
            pltpu.SemaphoreType.DMA,
            pltpu.SemaphoreType.DMA,
        ],
    )
    def sc_gather2(tab_a, tab_b, idx_a, idx_b, x0, x1, x2,
                   out_a, out_b, geo_out,
                   ia_v, ib_v, buf_a0, buf_a1, buf_b0, buf_b1,
                   x0_v, x1_v, x2_v, d2_v, za_v, zb_v,
                   s_a0, s_a1, s_b0, s_b1):
        wid = lax.axis_index("s") * 2 + lax.axis_index("c")
        pltpu.sync_copy(idx_a.at[wid], ia_v)
        pltpu.sync_copy(idx_b.at[wid], ib_v)
        pltpu.sync_copy(x0, x0_v)
        pltpu.sync_copy(x1, x1_v)
        pltpu.sync_copy(x2, x2_v)
        buf_a = (buf_a0, buf_a1)
        buf_b = (buf_b0, buf_b1)
        sem_a = (s_a0, s_a1)
        sem_b = (s_b0, s_b1)
        base0 = wid * EPW

        def start(c, b):
            pltpu.async_copy(tab_a.at[ia_v.at[c]], buf_a[b], sem_a[b])
            pltpu.async_copy(tab_b.at[ib_v.at[c]], buf_b[b], sem_b[b])

        def geom(c):
            @pl.loop(0, CH // 16)
            def _(j):
                iav = ia_v[c, pl.ds(j * 16, 16)]
                ibv = ib_v[c, pl.ds(j * 16, 16)]
                xa0 = plsc.load_gather(x0_v, [iav])
                xb0 = plsc.load_gather(x0_v, [ibv])
                xa1 = plsc.load_gather(x1_v, [iav])
                xb1 = plsc.load_gather(x1_v, [ibv])
                d0 = xa0 - xb0
                d1 = xa1 - xb1
                d2_v[pl.ds(j * 16, 16)] = d0 * d0 + d1 * d1
                za_v[pl.ds(j * 16, 16)] = plsc.load_gather(x2_v, [iav])
                zb_v[pl.ds(j * 16, 16)] = plsc.load_gather(x2_v, [ibv])
            t0 = base0 + c * CH
            blk = t0 // BE
            off = t0 % BE
            gbase = blk * (8 * BE) + off
            pltpu.sync_copy(d2_v, geo_out.at[pl.ds(gbase, CH)])
            pltpu.sync_copy(za_v, geo_out.at[pl.ds(gbase + BE, CH)])
            pltpu.sync_copy(zb_v, geo_out.at[pl.ds(gbase + 2 * BE, CH)])

        def finish(c, b):
            pltpu.make_async_copy(tab_a.at[ia_v.at[c]], buf_a[b], sem_a[b]).wait()
            pltpu.make_async_copy(tab_b.at[ib_v.at[c]], buf_b[b], sem_b[b]).wait()
            pltpu.sync_copy(buf_a[b], out_a.at[pl.ds(base0 + c * CH, CH)])
            pltpu.sync_copy(buf_b[b], out_b.at[pl.ds(base0 + c * CH, CH)])

        start(0, 0)

        @pl.loop(0, NCH, step=2)
        def _(c):
            start(c + 1, 1)
            geom(c)
            finish(c, 0)

            @pl.when(c + 2 < NCH)
            def _():
                start(c + 2, 0)

            geom(c + 1)
            finish(c + 1, 1)

    return sc_gather2


def _sc_gather2(tab_a, tab_b, idx_a, idx_b, x0, x1, x2):
    return _get_sc_gather2()(tab_a, tab_b, idx_a, idx_b, x0, x1, x2)


@functools.cache
def _get_sc_scatter():
    mesh = plsc.VectorSubcoreMesh(core_axis_name="c", subcore_axis_name="s")

    @functools.partial(
        pl.kernel,
        mesh=mesh,
        out_type=jax.ShapeDtypeStruct((2, NACC, DH), jnp.float32),
        scratch_types=[
            pltpu.VMEM((NCH, CH), jnp.int32),
            pltpu.VMEM((CH, DH), jnp.float32),
            pltpu.VMEM((CH, DH), jnp.float32),
            pltpu.VMEM_SHARED((NACC, DH), jnp.float32),
            pltpu.SemaphoreType.DMA,
            pltpu.SemaphoreType.DMA,
        ],
    )
    def sc_scatter(e_hbm, idx_d, z128, agg_out,
                   id_v, eb0, eb1, acc, s0, s1):
        cid = lax.axis_index("c")
        sid = lax.axis_index("s")
        wid = sid * 2 + cid
        rows = NACC // 16
        pltpu.sync_copy(idx_d.at[wid], id_v)
        pltpu.sync_copy(z128, acc.at[pl.ds(sid * rows, rows)])
        plsc.subcore_barrier()

        base0 = wid * EPW
        ebuf = (eb0, eb1)
        sems = (s0, s1)

        def start(c, b):
            pltpu.async_copy(e_hbm.at[pl.ds(base0 + c * CH, CH)], ebuf[b],
                             sems[b])

        def finish(c, b):
            pltpu.make_async_copy(
                e_hbm.at[pl.ds(base0 + c * CH, CH)], ebuf[b], sems[b]).wait()
            pltpu.sync_copy(ebuf[b], acc.at[id_v.at[c]], add=True)

        start(0, 0)

        @pl.loop(0, NCH, step=2)
        def _(c):
            start(c + 1, 1)
            finish(c, 0)

            @pl.when(c + 2 < NCH)
            def _():
                start(c + 2, 0)

            finish(c + 1, 1)

        plsc.subcore_barrier()
        pltpu.sync_copy(acc.at[pl.ds(sid * rows, rows)],
                        agg_out.at[cid, pl.ds(sid * rows, rows)])

    return sc_scatter


def _sc_scatter(e_hbm, idx_d, z128):
    return _get_sc_scatter()(e_hbm, idx_d, z128)


def _prologue_body(x_ref, w1_ref, b1_ref, w2_ref, b2_ref, wa_ref, wb_ref,
                   b1m_ref, h_ref, ts_ref, td_ref):
    x = x_ref[...]
    h1 = jnp.maximum(
        jnp.dot(x, w1_ref[...], preferred_element_type=jnp.float32,
                precision=_P_HI) + b1_ref[...], 0.0)
    h = jnp.dot(h1, w2_ref[...], preferred_element_type=jnp.float32,
                precision=_P_HI) + b2_ref[...]
    h_ref[...] = h
    ts_ref[...] = jnp.dot(h, wa_ref[...], preferred_element_type=jnp.float32,
                          precision=_P_HI)
    td_ref[...] = (jnp.dot(h, wb_ref[...], preferred_element_type=jnp.float32,
                           precision=_P_HI) + b1m_ref[...])


def _edge_body(mu_ref, w1c_ref, w2_ref, b2_ref, hs_ref, hd_ref, geo_ref,
               e_ref):
    geo = jnp.transpose(geo_ref[0])
    r = jnp.sqrt(geo[:, 0:1] + 1e-8)
    t = r - mu_ref[...]
    rbf = jnp.exp(-(t * t) / C2S2)
    g = jnp.dot(rbf, w1c_ref[...], preferred_element_type=jnp.float32)
    m1 = jnp.maximum(hs_ref[...] + hd_ref[...] + g, 0.0)
    e_ref[...] = jnp.dot(m1, w2_ref[...],
                         preferred_element_type=jnp.float32) + b2_ref[...]


def _node_body(h_ref, a0_ref, a1_ref,
               pw1a_ref, pw1b_ref, pb1_ref,
               pw2_ref, pb2_ref, lng_ref, lnb_ref, ea_ref, eb_ref, eb1_ref,
               au_ref, av_ref):
    h = h_ref[...]
    agg = a0_ref[...] + a1_ref[...]
    t1 = jnp.maximum(
        jnp.dot(h, pw1a_ref[...], preferred_element_type=jnp.float32,
                precision=_P_HI)
        + jnp.dot(agg, pw1b_ref[...], preferred_element_type=jnp.float32,
                  precision=_P_HI)
        + pb1_ref[...], 0.0)
    h_up = jnp.dot(t1, pw2_ref[...], preferred_element_type=jnp.float32,
                   precision=_P_HI) + pb2_ref[...]
    y = h + h_up
    mu = jnp.mean(y, axis=1, keepdims=True)
    var = jnp.mean((y - mu) ** 2, axis=1, keepdims=True)
    hn = (y - mu) / jnp.sqrt(var + 1e-5) * lng_ref[...] + lnb_ref[...]
    au_ref[...] = jnp.dot(hn, ea_ref[...], preferred_element_type=jnp.float32,
                          precision=_P_HI)
    av_ref[...] = (jnp.dot(hn, eb_ref[...], preferred_element_type=jnp.float32,
                           precision=_P_HI) + eb1_ref[...])


def _pair_body(w4_ref, w2r_ref, b2_ref, pu_ref, pv_ref, geo_ref, o_ref):
    geo = jnp.transpose(geo_ref[0])
    dist = jnp.sqrt(geo[:, 0:1] + 1e-8)
    ru = geo[:, 1:2]
    rv = geo[:, 2:3]
    rp = jnp.concatenate([ru, rv, dist, jnp.abs(ru - rv)], axis=1)
    g = jnp.dot(rp, w4_ref[...], preferred_element_type=jnp.float32)
    z = jnp.maximum(pu_ref[...] + pv_ref[...] + g, 0.0)
    o_ref[...] = jnp.dot(z, w2r_ref[...],
                         preferred_element_type=jnp.float32) + b2_ref[...]


def _sds(shape):
    return jax.ShapeDtypeStruct(shape, jnp.float32)


_prologue = pl.pallas_call(
    _prologue_body,
    grid=(NPAD // BN,),
    in_specs=[
        pl.BlockSpec((BN, 3), lambda i: (i, 0)),
        pl.BlockSpec((3, DH), lambda i: (0, 0)),
        pl.BlockSpec((1, DH), lambda i: (0, 0)),
        pl.BlockSpec((DH, DH), lambda i: (0, 0)),
        pl.BlockSpec((1, DH), lambda i: (0, 0)),
        pl.BlockSpec((DH, DH), lambda i: (0, 0)),
        pl.BlockSpec((DH, DH), lambda i: (0, 0)),
        pl.BlockSpec((1, DH), lambda i: (0, 0)),
    ],
    out_specs=[
        pl.BlockSpec((BN, DH), lambda i: (i, 0)),
        pl.BlockSpec((BN, DH), lambda i: (i, 0)),
        pl.BlockSpec((BN, DH), lambda i: (i, 0)),
    ],
    out_shape=(_sds((NPAD, DH)), _sds((NPAD, DH)), _sds((NPAD, DH))),
)

_edge_mlp = pl.pallas_call(
    _edge_body,
    grid=(NBE,),
    in_specs=[
        pl.BlockSpec((1, NRBF), lambda i: (0, 0)),
        pl.BlockSpec((NRBF, DH), lambda i: (0, 0)),
        pl.BlockSpec((DH, DH), lambda i: (0, 0)),
        pl.BlockSpec((1, DH), lambda i: (0, 0)),
        pl.BlockSpec((BE, DH), lambda i: (i, 0)),
        pl.BlockSpec((BE, DH), lambda i: (i, 0)),
        pl.BlockSpec((1, 8, BE), lambda i: (i, 0, 0)),
    ],
    out_specs=pl.BlockSpec((BE, DH), lambda i: (i, 0)),
    out_shape=_sds((EPAD, DH)),
)

_node_update = pl.pallas_call(
    _node_body,
    grid=(NPAD // BN,),
    in_specs=[
        pl.BlockSpec((BN, DH), lambda i: (i, 0)),
        pl.BlockSpec((BN, DH), lambda i: (i, 0)),
        pl.BlockSpec((BN, DH), lambda i: (i, 0)),
        pl.BlockSpec((DH, DH), lambda i: (0, 0)),
        pl.BlockSpec((DH, DH), lambda i: (0, 0)),
        pl.BlockSpec((1, DH), lambda i: (0, 0)),
        pl.BlockSpec((DH, DH), lambda i: (0, 0)),
        pl.BlockSpec((1, DH), lambda i: (0, 0)),
        pl.BlockSpec((1, DH), lambda i: (0, 0)),
        pl.BlockSpec((1, DH), lambda i: (0, 0)),
        pl.BlockSpec((DH, DH), lambda i: (0, 0)),
        pl.BlockSpec((DH, DH), lambda i: (0, 0)),
        pl.BlockSpec((1, DH), lambda i: (0, 0)),
    ],
    out_specs=[
        pl.BlockSpec((BN, DH), lambda i: (i, 0)),
        pl.BlockSpec((BN, DH), lambda i: (i, 0)),
    ],
    out_shape=(_sds((NPAD, DH)), _sds((NPAD, DH))),
)

_pair_head = pl.pallas_call(
    _pair_body,
    grid=(NBE,),
    in_specs=[
        pl.BlockSpec((4, DH), lambda i: (0, 0)),
        pl.BlockSpec((DH, 1), lambda i: (0, 0)),
        pl.BlockSpec((1, 1), lambda i: (0, 0)),
        pl.BlockSpec((BE, DH), lambda i: (i, 0)),
        pl.BlockSpec((BE, DH), lambda i: (i, 0)),
        pl.BlockSpec((1, 8, BE), lambda i: (i, 0, 0)),
    ],
    out_specs=pl.BlockSpec((BE, 1), lambda i: (i, 0)),
    out_shape=_sds((EPAD, 1)),
)


def kernel(xyr01, msg_edge_index, cand_pairs_uv,
           node_in_w1, node_in_b1, node_in_w2, node_in_b2,
           phi_m_w1, phi_m_b1, phi_m_w2, phi_m_b2,
           phi_h_w1, phi_h_b1, phi_h_w2, phi_h_b2,
           ln_g, ln_b,
           eh_w1, eh_b1, eh_w2, eh_b2):
    f32 = jnp.float32
    i32 = jnp.int32
    x = xyr01.astype(f32)
    xp = jnp.zeros((NPAD, 3), f32).at[:N].set(x)
    x0 = xp[:, 0]
    x1 = xp[:, 1]
    x2 = xp[:, 2]

    src = msg_edge_index[0].astype(i32)
    dst = msg_edge_index[1].astype(i32)
    u = cand_pairs_uv[:, 0].astype(i32)
    v = cand_pairs_uv[:, 1].astype(i32)
    pad = EPAD - E
    zpad_i = jnp.zeros((pad,), i32)
    srcp = jnp.concatenate([src, zpad_i]).reshape(NW, NCH, CH)
    dstp = jnp.concatenate([dst, jnp.full((pad,), N, i32)]).reshape(NW, NCH, CH)
    up = jnp.concatenate([u, zpad_i]).reshape(NW, NCH, CH)
    vp = jnp.concatenate([v, zpad_i]).reshape(NW, NCH, CH)

    wa = phi_m_w1[0:DH]
    wb = phi_m_w1[DH:2 * DH]
    w1c = phi_m_w1[2 * DH:]
    mu_row = jnp.linspace(0.0, RMAX, NRBF, dtype=f32).reshape(1, NRBF)

    h, ts_tab, td_tab = _prologue(
        xp, node_in_w1, node_in_b1.reshape(1, DH), node_in_w2,
        node_in_b2.reshape(1, DH), wa, wb, phi_m_b1.reshape(1, DH))

    hs, hd, geo_e = _sc_gather2(ts_tab, td_tab, srcp, dstp, x0, x1, x2)
    geo_e = geo_e.reshape(NBE, 8, BE)

    e = _edge_mlp(mu_row, w1c, phi_m_w2, phi_m_b2.reshape(1, DH),
                  hs, hd, geo_e)

    z128 = jnp.zeros((NACC // 16, DH), f32)
    agg2 = _sc_scatter(e, dstp, z128)

    npad2 = ((0, 0), (0, NPAD - NACC), (0, 0))
    agg2 = jnp.pad(agg2, npad2)

    au_tab, av_tab = _node_update(
        h, agg2[0], agg2[1],
        phi_h_w1[0:DH], phi_h_w1[DH:2 * DH], phi_h_b1.reshape(1, DH),
        phi_h_w2, phi_h_b2.reshape(1, DH),
        ln_g.reshape(1, DH), ln_b.reshape(1, DH),
        eh_w1[0:DH], eh_w1[DH:2 * DH], eh_b1.reshape(1, DH))

    pu, pv, geo_p = _sc_gather2(au_tab, av_tab, up, vp, x0, x1, x2)
    geo_p = geo_p.reshape(NBE, 8, BE)

    out = _pair_head(eh_w1[2 * DH:], eh_w2,
                     eh_b2.reshape(1, 1), pu, pv, geo_p)
    return out[:P, 0]


# speedup vs baseline: 19.6641x; 1.0543x over previous
import dataclasses
import functools

import jax
import jax.numpy as jnp
from jax import lax
from jax.experimental import pallas as pl
from jax.experimental.pallas import tpu as pltpu
from jax.experimental.pallas import tpu_sc as plsc

N = 10000
E = 320000
P = 320000
DH = 128
NRBF = 16
RMAX = 1.4142135623730951
C2S2 = 2.0 * (RMAX / 15.0) * (RMAX / 15.0)
INV2S2 = 56.25

NPAD = 10240
NW = 32
CH = 128
NCH = 80
EPW = CH * NCH
EPAD = NW * EPW
NACC = 10112
BN = 1024
BE = 2048
NBE = EPAD // BE
GPB = BE // CH

_P_HI = None


def _sc_compiler_params():
    cp = pltpu.CompilerParams()
    if "needs_layout_passes" in pltpu.CompilerParams.__dataclass_fields__:
        cp = dataclasses.replace(cp, needs_layout_passes=False)
    return cp


@functools.cache
def _get_sc_gather2():
    mesh = plsc.VectorSubcoreMesh(core_axis_name="c", subcore_axis_name="s")

    @functools.partial(
        pl.kernel,
        mesh=mesh,
        compiler_params=_sc_compiler_params(),
        out_type=(jax.ShapeDtypeStruct((EPAD, DH), jnp.float32),
                  jax.ShapeDtypeStruct((NBE * 8 * BE,), jnp.float32)),
        scratch_types=[
            pltpu.VMEM((NCH, CH), jnp.int32),
            pltpu.VMEM((NCH, CH), jnp.int32),
            pltpu.VMEM((CH, DH), jnp.float32),
            pltpu.VMEM((CH, DH), jnp.float32),
            pltpu.VMEM((CH, DH), jnp.float32),
            pltpu.VMEM((CH, DH), jnp.float32),
            pltpu.VMEM((NPAD,), jnp.float32),
            pltpu.VMEM((NPAD,), jnp.float32),
            pltpu.VMEM((NPAD,), jnp.float32),
            pltpu.VMEM((CH,), jnp.float32),
            pltpu.VMEM((CH,), jnp.float32),
            pltpu.VMEM((CH,), jnp.float32),
            pltpu.VMEM((CH // 2,), jnp.int32),
            pltpu.VMEM_SHARED((16 * CH // 2, DH), jnp.float32),
            pltpu.SemaphoreType.DMA,
            pltpu.SemaphoreType.DMA,
            pltpu.SemaphoreType.DMA,
            pltpu.SemaphoreType.DMA,
        ],
    )
    def sc_gather2(tab_a, tab_b, idx_a, idx_b, x0, x1, x2,
                   out_s, geo_out,
                   ia_v, ib_v, buf_a0, buf_a1, buf_b0, buf_b1,
                   x0_v, x1_v, x2_v, d2_v, za_v, zb_v, ident_v, sbuf,
                   s_a0, s_a1, s_b0, s_b1):
        wid = lax.axis_index("s") * 2 + lax.axis_index("c")
        pltpu.sync_copy(idx_a.at[wid], ia_v)
        pltpu.sync_copy(idx_b.at[wid], ib_v)
        pltpu.sync_copy(x0, x0_v)
        pltpu.sync_copy(x1, x1_v)
        pltpu.sync_copy(x2, x2_v)

        sid = lax.axis_index("s")
        half = CH // 2

        @pl.loop(0, half // 16)
        def _(j):
            ident_v[pl.ds(j * 16, 16)] = (lax.iota(jnp.int32, 16) + j * 16
                                          + sid * half)
        buf_a = (buf_a0, buf_a1)
        buf_b = (buf_b0, buf_b1)
        sem_a = (s_a0, s_a1)
        sem_b = (s_b0, s_b1)
        base0 = wid * EPW

        def start(c, b):
            pltpu.async_copy(tab_a.at[ia_v.at[c]], buf_a[b], sem_a[b])
            pltpu.async_copy(tab_b.at[ib_v.at[c]], buf_b[b], sem_b[b])

        def geom(c):
            @pl.loop(0, CH // 16)
            def _(j):
                iav = ia_v[c, pl.ds(j * 16, 16)]
                ibv = ib_v[c, pl.ds(j * 16, 16)]
                xa0 = plsc.load_gather(x0_v, [iav])
                xb0 = plsc.load_gather(x0_v, [ibv])
                xa1 = plsc.load_gather(x1_v, [iav])
                xb1 = plsc.load_gather(x1_v, [ibv])
                d0 = xa0 - xb0
                d1 = xa1 - xb1
                d2_v[pl.ds(j * 16, 16)] = d0 * d0 + d1 * d1
                za_v[pl.ds(j * 16, 16)] = plsc.load_gather(x2_v, [iav])
                zb_v[pl.ds(j * 16, 16)] = plsc.load_gather(x2_v, [ibv])
            t0 = base0 + c * CH
            blk = t0 // BE
            off = t0 % BE
            gbase = blk * (8 * BE) + off
            pltpu.sync_copy(d2_v, geo_out.at[pl.ds(gbase, CH)])
            pltpu.sync_copy(za_v, geo_out.at[pl.ds(gbase + BE, CH)])
            pltpu.sync_copy(zb_v, geo_out.at[pl.ds(gbase + 2 * BE, CH)])

        def finish(c, b):
            pltpu.make_async_copy(tab_a.at[ia_v.at[c]], buf_a[b], sem_a[b]).wait()
            pltpu.make_async_copy(tab_b.at[ib_v.at[c]], buf_b[b], sem_b[b]).wait()
            for hh in (0, 1):
                pltpu.sync_copy(buf_a[b].at[pl.ds(hh * half, half)],
                                sbuf.at[pl.ds(sid * half, half)])
                pltpu.sync_copy(buf_b[b].at[pl.ds(hh * half, half)],
                                sbuf.at[ident_v], add=True)
                pltpu.sync_copy(
                    sbuf.at[pl.ds(sid * half, half)],
                    out_s.at[pl.ds(base0 + c * CH + hh * half, half)])

        start(0, 0)

        @pl.loop(0, NCH, step=2)
        def _(c):
            start(c + 1, 1)
            geom(c)
            finish(c, 0)

            @pl.when(c + 2 < NCH)
            def _():
                start(c + 2, 0)

            geom(c + 1)
            finish(c + 1, 1)

    return sc_gather2


def _sc_gather2(tab_a, tab_b, idx_a, idx_b, x0, x1, x2):
    return _get_sc_gather2()(tab_a, tab_b, idx_a, idx_b, x0, x1, x2)


@functools.cache
def _get_sc_scatter():
    mesh = plsc.VectorSubcoreMesh(core_axis_name="c", subcore_axis_name="s")

    @functools.partial(
        pl.kernel,
        mesh=mesh,
        out_type=jax.ShapeDtypeStruct((2, NACC, DH), jnp.float32),
        scratch_types=[
            pltpu.VMEM((NCH, CH), jnp.int32),
            pltpu.VMEM((CH, DH), jnp.float32),
            pltpu.VMEM((CH, DH), jnp.float32),
            pltpu.VMEM_SHARED((NACC, DH), jnp.float32),
            pltpu.SemaphoreType.DMA,
            pltpu.SemaphoreType.DMA,
        ],
    )
    def sc_scatter(e_hbm, idx_d, z128, agg_out,
                   id_v, eb0, eb1, acc, s0, s1):
        cid = lax.axis_index("c")
        sid = lax.axis_index("s")
        wid = sid * 2 + cid
        rows = NACC // 16
        pltpu.sync_copy(idx_d.at[wid], id_v)
        pltpu.sync_copy(z128, acc.at[pl.ds(sid * rows, rows)])
        plsc.subcore_barrier()

        base0 = wid * EPW
        ebuf = (eb0, eb1)
        sems = (s0, s1)

        def start(c, b):
            pltpu.async_copy(e_hbm.at[pl.ds(base0 + c * CH, CH)], ebuf[b],
                             sems[b])

        def finish(c, b):
            pltpu.make_async_copy(
                e_hbm.at[pl.ds(base0 + c * CH, CH)], ebuf[b], sems[b]).wait()
            pltpu.sync_copy(ebuf[b], acc.at[id_v.at[c]], add=True)

        start(0, 0)

        @pl.loop(0, NCH, step=2)
        def _(c):
            start(c + 1, 1)
            finish(c, 0)

            @pl.when(c + 2 < NCH)
            def _():
                start(c + 2, 0)

            finish(c + 1, 1)

        plsc.subcore_barrier()
        pltpu.sync_copy(acc.at[pl.ds(sid * rows, rows)],
                        agg_out.at[cid, pl.ds(sid * rows, rows)])

    return sc_scatter


def _sc_scatter(e_hbm, idx_d, z128):
    return _get_sc_scatter()(e_hbm, idx_d, z128)


def _prologue_body(x_ref, w1_ref, b1_ref, w2_ref, b2_ref, wa_ref, wb_ref,
                   b1m_ref, h_ref, ts_ref, td_ref):
    x = x_ref[...]
    h1 = jnp.maximum(
        jnp.dot(x, w1_ref[...], preferred_element_type=jnp.float32,
                precision=_P_HI) + b1_ref[...], 0.0)
    h = jnp.dot(h1, w2_ref[...], preferred_element_type=jnp.float32,
                precision=_P_HI) + b2_ref[...]
    h_ref[...] = h
    ts_ref[...] = jnp.dot(h, wa_ref[...], preferred_element_type=jnp.float32,
                          precision=_P_HI)
    td_ref[...] = (jnp.dot(h, wb_ref[...], preferred_element_type=jnp.float32,
                           precision=_P_HI) + b1m_ref[...])


def _edge_body(mu_ref, w1c_ref, w2_ref, b2_ref, s_ref, geo_ref, e_ref):
    geo = jnp.transpose(geo_ref[0])
    r = jnp.sqrt(geo[:, 0:1] + 1e-8)
    t = r - mu_ref[...]
    rbf = jnp.exp(-(t * t) / C2S2)
    g = jnp.dot(rbf, w1c_ref[...], preferred_element_type=jnp.float32)
    m1 = jnp.maximum(s_ref[...] + g, 0.0)
    e_ref[...] = jnp.dot(m1, w2_ref[...],
                         preferred_element_type=jnp.float32) + b2_ref[...]


def _node_body(h_ref, a0_ref, a1_ref,
               pw1a_ref, pw1b_ref, pb1_ref,
               pw2_ref, pb2_ref, lng_ref, lnb_ref, ea_ref, eb_ref, eb1_ref,
               au_ref, av_ref):
    h = h_ref[...]
    agg = a0_ref[...] + a1_ref[...]
    t1 = jnp.maximum(
        jnp.dot(h, pw1a_ref[...], preferred_element_type=jnp.float32,
                precision=_P_HI)
        + jnp.dot(agg, pw1b_ref[...], preferred_element_type=jnp.float32,
                  precision=_P_HI)
        + pb1_ref[...], 0.0)
    h_up = jnp.dot(t1, pw2_ref[...], preferred_element_type=jnp.float32,
                   precision=_P_HI) + pb2_ref[...]
    y = h + h_up
    mu = jnp.mean(y, axis=1, keepdims=True)
    var = jnp.mean((y - mu) ** 2, axis=1, keepdims=True)
    hn = (y - mu) / jnp.sqrt(var + 1e-5) * lng_ref[...] + lnb_ref[...]
    au_ref[...] = jnp.dot(hn, ea_ref[...], preferred_element_type=jnp.float32,
                          precision=_P_HI)
    av_ref[...] = (jnp.dot(hn, eb_ref[...], preferred_element_type=jnp.float32,
                           precision=_P_HI) + eb1_ref[...])


def _pair_body(w4_ref, w2r_ref, b2_ref, s_ref, geo_ref, o_ref):
    geo = jnp.transpose(geo_ref[0])
    dist = jnp.sqrt(geo[:, 0:1] + 1e-8)
    ru = geo[:, 1:2]
    rv = geo[:, 2:3]
    rp = jnp.concatenate([ru, rv, dist, jnp.abs(ru - rv)], axis=1)
    g = jnp.dot(rp, w4_ref[...], preferred_element_type=jnp.float32)
    z = jnp.maximum(s_ref[...] + g, 0.0)
    o_ref[...] = jnp.dot(z, w2r_ref[...],
                         preferred_element_type=jnp.float32) + b2_ref[...]


def _sds(shape):
    return jax.ShapeDtypeStruct(shape, jnp.float32)


_prologue = pl.pallas_call(
    _prologue_body,
    grid=(NPAD // BN,),
    in_specs=[
        pl.BlockSpec((BN, 3), lambda i: (i, 0)),
        pl.BlockSpec((3, DH), lambda i: (0, 0)),
        pl.BlockSpec((1, DH), lambda i: (0, 0)),
        pl.BlockSpec((DH, DH), lambda i: (0, 0)),
        pl.BlockSpec((1, DH), lambda i: (0, 0)),
        pl.BlockSpec((DH, DH), lambda i: (0, 0)),
        pl.BlockSpec((DH, DH), lambda i: (0, 0)),
        pl.BlockSpec((1, DH), lambda i: (0, 0)),
    ],
    out_specs=[
        pl.BlockSpec((BN, DH), lambda i: (i, 0)),
        pl.BlockSpec((BN, DH), lambda i: (i, 0)),
        pl.BlockSpec((BN, DH), lambda i: (i, 0)),
    ],
    out_shape=(_sds((NPAD, DH)), _sds((NPAD, DH)), _sds((NPAD, DH))),
)

_edge_mlp = pl.pallas_call(
    _edge_body,
    grid=(NBE,),
    in_specs=[
        pl.BlockSpec((1, NRBF), lambda i: (0, 0)),
        pl.BlockSpec((NRBF, DH), lambda i: (0, 0)),
        pl.BlockSpec((DH, DH), lambda i: (0, 0)),
        pl.BlockSpec((1, DH), lambda i: (0, 0)),
        pl.BlockSpec((BE, DH), lambda i: (i, 0)),
        pl.BlockSpec((1, 8, BE), lambda i: (i, 0, 0)),
    ],
    out_specs=pl.BlockSpec((BE, DH), lambda i: (i, 0)),
    out_shape=_sds((EPAD, DH)),
)

_node_update = pl.pallas_call(
    _node_body,
    grid=(NPAD // BN,),
    in_specs=[
        pl.BlockSpec((BN, DH), lambda i: (i, 0)),
        pl.BlockSpec((BN, DH), lambda i: (i, 0)),
        pl.BlockSpec((BN, DH), lambda i: (i, 0)),
        pl.BlockSpec((DH, DH), lambda i: (0, 0)),
        pl.BlockSpec((DH, DH), lambda i: (0, 0)),
        pl.BlockSpec((1, DH), lambda i: (0, 0)),
        pl.BlockSpec((DH, DH), lambda i: (0, 0)),
        pl.BlockSpec((1, DH), lambda i: (0, 0)),
        pl.BlockSpec((1, DH), lambda i: (0, 0)),
        pl.BlockSpec((1, DH), lambda i: (0, 0)),
        pl.BlockSpec((DH, DH), lambda i: (0, 0)),
        pl.BlockSpec((DH, DH), lambda i: (0, 0)),
        pl.BlockSpec((1, DH), lambda i: (0, 0)),
    ],
    out_specs=[
        pl.BlockSpec((BN, DH), lambda i: (i, 0)),
        pl.BlockSpec((BN, DH), lambda i: (i, 0)),
    ],
    out_shape=(_sds((NPAD, DH)), _sds((NPAD, DH))),
)

_pair_head = pl.pallas_call(
    _pair_body,
    grid=(NBE,),
    in_specs=[
        pl.BlockSpec((4, DH), lambda i: (0, 0)),
        pl.BlockSpec((DH, 1), lambda i: (0, 0)),
        pl.BlockSpec((1, 1), lambda i: (0, 0)),
        pl.BlockSpec((BE, DH), lambda i: (i, 0)),
        pl.BlockSpec((1, 8, BE), lambda i: (i, 0, 0)),
    ],
    out_specs=pl.BlockSpec((BE, 1), lambda i: (i, 0)),
    out_shape=_sds((EPAD, 1)),
)


def kernel(xyr01, msg_edge_index, cand_pairs_uv,
           node_in_w1, node_in_b1, node_in_w2, node_in_b2,
           phi_m_w1, phi_m_b1, phi_m_w2, phi_m_b2,
           phi_h_w1, phi_h_b1, phi_h_w2, phi_h_b2,
           ln_g, ln_b,
           eh_w1, eh_b1, eh_w2, eh_b2):
    f32 = jnp.float32
    i32 = jnp.int32
    x = xyr01.astype(f32)
    xp = jnp.zeros((NPAD, 3), f32).at[:N].set(x)
    x0 = xp[:, 0]
    x1 = xp[:, 1]
    x2 = xp[:, 2]

    src = msg_edge_index[0].astype(i32)
    dst = msg_edge_index[1].astype(i32)
    u = cand_pairs_uv[:, 0].astype(i32)
    v = cand_pairs_uv[:, 1].astype(i32)
    pad = EPAD - E
    zpad_i = jnp.zeros((pad,), i32)
    srcp = jnp.concatenate([src, zpad_i]).reshape(NW, NCH, CH)
    dstp = jnp.concatenate([dst, jnp.full((pad,), N, i32)]).reshape(NW, NCH, CH)
    up = jnp.concatenate([u, zpad_i]).reshape(NW, NCH, CH)
    vp = jnp.concatenate([v, zpad_i]).reshape(NW, NCH, CH)

    wa = phi_m_w1[0:DH]
    wb = phi_m_w1[DH:2 * DH]
    w1c = phi_m_w1[2 * DH:]
    mu_row = jnp.linspace(0.0, RMAX, NRBF, dtype=f32).reshape(1, NRBF)

    h, ts_tab, td_tab = _prologue(
        xp, node_in_w1, node_in_b1.reshape(1, DH), node_in_w2,
        node_in_b2.reshape(1, DH), wa, wb, phi_m_b1.reshape(1, DH))

    s_e, geo_e = _sc_gather2(ts_tab, td_tab, srcp, dstp, x0, x1, x2)
    geo_e = geo_e.reshape(NBE, 8, BE)

    e = _edge_mlp(mu_row, w1c, phi_m_w2, phi_m_b2.reshape(1, DH),
                  s_e, geo_e)

    z128 = jnp.zeros((NACC // 16, DH), f32)
    agg2 = _sc_scatter(e, dstp, z128)

    npad2 = ((0, 0), (0, NPAD - NACC), (0, 0))
    agg2 = jnp.pad(agg2, npad2)

    au_tab, av_tab = _node_update(
        h, agg2[0], agg2[1],
        phi_h_w1[0:DH], phi_h_w1[DH:2 * DH], phi_h_b1.reshape(1, DH),
        phi_h_w2, phi_h_b2.reshape(1, DH),
        ln_g.reshape(1, DH), ln_b.reshape(1, DH),
        eh_w1[0:DH], eh_w1[DH:2 * DH], eh_b1.reshape(1, DH))

    s_p, geo_p = _sc_gather2(au_tab, av_tab, up, vp, x0, x1, x2)
    geo_p = geo_p.reshape(NBE, 8, BE)

    out = _pair_head(eh_w1[2 * DH:], eh_w2,
                     eh_b2.reshape(1, 1), s_p, geo_p)
    return out[:P, 0]
